# baseline (speedup 1.0000x reference)
import functools

import jax
import jax.numpy as jnp
from jax import lax
from jax.experimental import pallas as pl
from jax.experimental.pallas import tpu as pltpu
from jax.experimental.pallas import tpu_sc as plsc

N_FIELDS = 26
VOCAB = 100001
DIM = 32
BATCH = 4096

NUM_CORES = 2
NUM_SUBCORES = 16
NUM_WORKERS = NUM_CORES * NUM_SUBCORES
ROWS = BATCH * N_FIELDS
ROWS_PER_WORKER = ROWS // NUM_WORKERS
CHUNK = 128
NCHUNKS = ROWS_PER_WORKER // CHUNK
LANES = 16


def _sc_gather(idx_r, tables_flat):
    mesh = plsc.VectorSubcoreMesh(core_axis_name="c", subcore_axis_name="s")

    @functools.partial(
        pl.kernel,
        mesh=mesh,
        out_type=jax.ShapeDtypeStruct(
            (NUM_WORKERS, NCHUNKS, CHUNK, DIM), jnp.float32),
        scratch_types=[
            pltpu.VMEM((NCHUNKS, CHUNK), jnp.int32),
            pltpu.VMEM((NCHUNKS, CHUNK, DIM), jnp.float32),
            pltpu.SemaphoreType.DMA,
        ],
        compiler_params=pltpu.CompilerParams(use_tc_tiling_on_sc=False),
    )
    def k(idx_hbm, tab_hbm, out_hbm, idx_v, rows_v, sem):
        wid = lax.axis_index("s") * NUM_CORES + lax.axis_index("c")
        pltpu.sync_copy(idx_hbm.at[wid], idx_v)
        copies = []
        for j in range(NCHUNKS):
            rem = (j * CHUNK + lax.iota(jnp.int32, LANES)) % N_FIELDS

            def cbody(c, rem, j=j):
                sl = pl.ds(c * LANES, LANES)
                idx_v[j, sl] = idx_v[j, sl] + rem * VOCAB
                rem = rem + LANES
                return jnp.where(rem >= N_FIELDS, rem - N_FIELDS, rem)

            lax.fori_loop(0, CHUNK // LANES, cbody, rem)
            copies.append(
                pltpu.async_copy(tab_hbm.at[idx_v.at[j]], rows_v.at[j], sem))
        for cp in copies:
            cp.wait()
        pltpu.sync_copy(rows_v, out_hbm.at[wid])

    return k(idx_r, tables_flat)


def kernel(indices, tables):
    tables_flat = tables.reshape(N_FIELDS * VOCAB, DIM)
    idx_r = indices.reshape(NUM_WORKERS, NCHUNKS, CHUNK)
    out = _sc_gather(idx_r, tables_flat)
    return out.reshape(BATCH, N_FIELDS, DIM)


# speedup vs baseline: 2.5095x; 2.5095x over previous
import functools

import jax
import jax.numpy as jnp
from jax import lax
from jax.experimental import pallas as pl
from jax.experimental.pallas import tpu as pltpu
from jax.experimental.pallas import tpu_sc as plsc

N_FIELDS = 26
VOCAB = 100001
DIM = 32
BATCH = 4096
LANES = 16

NUM_CORES = 2
NUM_SUBCORES = 16
NUM_WORKERS = NUM_CORES * NUM_SUBCORES
BPW = BATCH // NUM_WORKERS


def kernel(indices, tables):
    mesh = plsc.VectorSubcoreMesh(core_axis_name="c", subcore_axis_name="s")

    @functools.partial(
        pl.kernel,
        mesh=mesh,
        out_type=jax.ShapeDtypeStruct((BATCH, N_FIELDS, DIM), jnp.float32),
        scratch_types=[
            pltpu.VMEM((BPW, N_FIELDS), jnp.int32),
            pltpu.VMEM((N_FIELDS, BPW), jnp.int32),
            pltpu.VMEM((N_FIELDS, BPW, DIM), jnp.float32),
            pltpu.SemaphoreType.DMA,
            pltpu.SemaphoreType.DMA,
        ],
        compiler_params=pltpu.CompilerParams(
            use_tc_tiling_on_sc=False, needs_layout_passes=False),
    )
    def k(idx_hbm, tab_hbm, out_hbm, idx_slab, idx_col, rows_v, gsem, osem):
        wid = lax.axis_index("s") * NUM_CORES + lax.axis_index("c")
        base = wid * BPW
        pltpu.sync_copy(idx_hbm.at[pl.ds(base, BPW)], idx_slab)
        for f in range(N_FIELDS):
            for c in range(BPW // LANES):
                rows16 = lax.iota(jnp.int32, LANES) + (c * LANES)
                cols16 = lax.iota(jnp.int32, LANES) * 0 + f
                idx_col[f, pl.ds(c * LANES, LANES)] = plsc.load_gather(
                    idx_slab, [rows16, cols16])
        gcopies = [
            pltpu.async_copy(
                tab_hbm.at[f].at[idx_col.at[f]], rows_v.at[f], gsem)
            for f in range(N_FIELDS)
        ]
        for cp in gcopies:
            cp.wait()
        ocopies = [
            pltpu.async_copy(
                rows_v.at[f], out_hbm.at[pl.ds(base, BPW), f], osem)
            for f in range(N_FIELDS)
        ]
        for cp in ocopies:
            cp.wait()

    return k(indices, tables)


# speedup vs baseline: 3.0820x; 1.2281x over previous
import functools

import jax
import jax.numpy as jnp
from jax import lax
from jax.experimental import pallas as pl
from jax.experimental.pallas import tpu as pltpu
from jax.experimental.pallas import tpu_sc as plsc

N_FIELDS = 26
VOCAB = 100001
DIM = 32
BATCH = 4096

NUM_CORES = 2
NUM_SUBCORES = 16
NUM_WORKERS = NUM_CORES * NUM_SUBCORES
PAIRS = N_FIELDS * DIM
PPW = PAIRS // NUM_WORKERS


def kernel(indices, tables):
    tab_t = jnp.transpose(tables, (0, 2, 1))
    idx_t = indices.T

    mesh = plsc.VectorSubcoreMesh(core_axis_name="c", subcore_axis_name="s")

    @functools.partial(
        pl.kernel,
        mesh=mesh,
        out_type=jax.ShapeDtypeStruct((N_FIELDS, DIM, BATCH), jnp.float32),
        scratch_types=[
            pltpu.VMEM((2, BATCH), jnp.int32),
            pltpu.VMEM((PPW, BATCH), jnp.float32),
            pltpu.SemaphoreType.DMA,
            pltpu.SemaphoreType.DMA,
        ],
        compiler_params=pltpu.CompilerParams(use_tc_tiling_on_sc=False),
    )
    def k(idx_hbm, tab_hbm, out_hbm, idx_v, gbuf, gsem, osem):
        wid = lax.axis_index("s") * NUM_CORES + lax.axis_index("c")
        p0 = wid * PPW
        f0 = p0 // DIM
        pltpu.sync_copy(idx_hbm.at[f0], idx_v.at[0])
        f1 = jnp.minimum(f0 + 1, N_FIELDS - 1)
        pltpu.sync_copy(idx_hbm.at[f1], idx_v.at[1])
        gcopies = []
        for j in range(PPW):
            p = p0 + j
            f = p // DIM
            d = p % DIM
            gcopies.append(
                pltpu.async_copy(
                    tab_hbm.at[f, d].at[idx_v.at[f - f0]],
                    gbuf.at[j], gsem))
        for cp in gcopies:
            cp.wait()
        ocopies = []
        for j in range(PPW):
            p = p0 + j
            ocopies.append(
                pltpu.async_copy(
                    gbuf.at[j], out_hbm.at[p // DIM, p % DIM], osem))
        for cp in ocopies:
            cp.wait()

    out_t = k(idx_t, tab_t)
    return jnp.transpose(out_t, (2, 0, 1))
